# Initial kernel scaffold; baseline (speedup 1.0000x reference)
import jax
import jax.numpy as jnp
from jax.experimental import pallas as pl


def kernel(x, edge_index, edge_attr, batch, node_emb, edge_emb, conv_W1, conv_b1, conv_W2, conv_b2, mlp_W1, mlp_b1, mlp_W2, mlp_b2, mlp_W3, mlp_b3):
    raise NotImplementedError("write your pallas kernel here")


# speedup vs baseline: 2.9281x; 2.9281x over previous
import functools

import jax
import jax.numpy as jnp
from jax import lax
from jax.experimental import pallas as pl
from jax.experimental.pallas import tpu as pltpu
from jax.experimental.pallas import tpu_sc as plsc

N = 10000
H = 128
NG = 64
NODE_VOCAB = 28
NLAYER = 3

NC = 2
NS = 16
NW = NC * NS
CH = 128
CPW = 80
EPW = CPW * CH
EPAD = NW * EPW
ACC_ROWS = 10112
ZRPT = ACC_ROWS // NS

SINGLE_TILE = False
CPW1 = (NW * CPW) // NC


@functools.lru_cache(maxsize=None)
def _get_msg_kernel():
    mesh = plsc.VectorSubcoreMesh(core_axis_name="c", subcore_axis_name="s")

    @functools.partial(
        pl.kernel,
        mesh=mesh,
        out_type=jax.ShapeDtypeStruct((NC * ACC_ROWS, H), jnp.float32),
        scratch_types=[
            pltpu.VMEM((CPW, CH), jnp.int32),
            pltpu.VMEM((CPW, CH), jnp.int32),
            pltpu.VMEM((CH, H), jnp.float32),
            pltpu.VMEM_SHARED((ACC_ROWS, H), jnp.float32),
            pltpu.SemaphoreType.DMA,
        ],
    )
    def _msg_kernel(src_hbm, dst_hbm, zero_hbm, h_hbm, out_hbm,
                    sidx, didx, rows, acc, sem):
        c = lax.axis_index("c")
        s = lax.axis_index("s")
        pltpu.sync_copy(zero_hbm, acc.at[pl.ds(s * ZRPT, ZRPT)])
        plsc.subcore_barrier()

        def body(j, carry):
            pltpu.async_copy(h_hbm.at[sidx.at[j]], rows, sem).wait()
            pltpu.sync_copy(rows, acc.at[didx.at[j]], add=True)
            return carry

        if SINGLE_TILE:
            @pl.when(s == 0)
            def _():
                def outer(g, carry):
                    pltpu.sync_copy(src_hbm.at[pl.ds(c * CPW1 + g * CPW, CPW)], sidx)
                    pltpu.sync_copy(dst_hbm.at[pl.ds(c * CPW1 + g * CPW, CPW)], didx)
                    lax.fori_loop(0, CPW, body, 0)
                    return carry
                lax.fori_loop(0, CPW1 // CPW, outer, 0)
        else:
            wid = s * NC + c
            pltpu.sync_copy(src_hbm.at[pl.ds(wid * CPW, CPW)], sidx)
            pltpu.sync_copy(dst_hbm.at[pl.ds(wid * CPW, CPW)], didx)
            lax.fori_loop(0, CPW, body, 0)
        plsc.subcore_barrier()
        base = c * ACC_ROWS + s * ZRPT
        pltpu.sync_copy(acc.at[pl.ds(s * ZRPT, ZRPT)],
                        out_hbm.at[pl.ds(base, ZRPT)])

    return _msg_kernel


def _embed_body(x_ref, emb_ref, o_ref):
    xv = x_ref[...]
    vid = lax.broadcasted_iota(jnp.int32, (NODE_VOCAB, N), 0)
    onehot = (vid == xv).astype(jnp.float32)
    o_ref[...] = lax.dot_general(
        onehot, emb_ref[...], (((0,), (0,)), ((), ())),
        precision=lax.Precision.HIGHEST,
        preferred_element_type=jnp.float32)


def _embed(x, node_emb):
    return pl.pallas_call(
        _embed_body,
        out_shape=jax.ShapeDtypeStruct((N, H), jnp.float32),
    )(x.reshape(1, N), node_emb)


def _dense_body(h_ref, m0_ref, m1_ref, w1_ref, b1_ref, w2_ref, b2_ref, o_ref):
    xv = h_ref[...] + m0_ref[...] + m1_ref[...]
    z = jnp.dot(xv, w1_ref[...], preferred_element_type=jnp.float32) + b1_ref[...]
    z = jnp.maximum(z, 0.0)
    o_ref[...] = jnp.dot(z, w2_ref[...], preferred_element_type=jnp.float32) + b2_ref[...]


def _dense(h, m0, m1, w1, b1, w2, b2):
    blk = 2000
    grid = N // blk
    row = lambda i: (i, 0)
    full = lambda i: (0, 0)
    return pl.pallas_call(
        _dense_body,
        grid=(grid,),
        in_specs=[
            pl.BlockSpec((blk, H), row),
            pl.BlockSpec((blk, H), row),
            pl.BlockSpec((blk, H), row),
            pl.BlockSpec((H, H), full),
            pl.BlockSpec((1, H), full),
            pl.BlockSpec((H, H), full),
            pl.BlockSpec((1, H), full),
        ],
        out_specs=pl.BlockSpec((blk, H), row),
        out_shape=jax.ShapeDtypeStruct((N, H), jnp.float32),
    )(h, m0, m1, w1, b1.reshape(1, H), w2, b2.reshape(1, H))


def _pool_body(h_ref, bat_ref, w1_ref, b1_ref, w2_ref, b2_ref, w3_ref, b3_ref,
               o_ref):
    bat = bat_ref[...]
    gid = lax.broadcasted_iota(jnp.int32, (NG, N), 0)
    onehot = (gid == bat).astype(jnp.float32)
    pooled = jnp.dot(onehot, h_ref[...], precision=lax.Precision.HIGHEST,
                     preferred_element_type=jnp.float32)
    a = jnp.maximum(jnp.dot(pooled, w1_ref[...], preferred_element_type=jnp.float32)
                    + b1_ref[...], 0.0)
    a = jnp.maximum(jnp.dot(a, w2_ref[...], preferred_element_type=jnp.float32)
                    + b2_ref[...], 0.0)
    o_ref[...] = jnp.dot(a, w3_ref[...], preferred_element_type=jnp.float32) + b3_ref[...]


def _pool_mlp(h, batch, w1, b1, w2, b2, w3, b3):
    return pl.pallas_call(
        _pool_body,
        out_shape=jax.ShapeDtypeStruct((NG, 1), jnp.float32),
    )(h, batch.reshape(1, N), w1, b1.reshape(1, H // 2), w2,
      b2.reshape(1, H // 4), w3, b3.reshape(1, 1))


def kernel(x, edge_index, edge_attr, batch, node_emb, edge_emb,
           conv_W1, conv_b1, conv_W2, conv_b2,
           mlp_W1, mlp_b1, mlp_W2, mlp_b2, mlp_W3, mlp_b3):
    src = edge_index[0]
    dst = edge_index[1]
    e = src.shape[0]
    pad = EPAD - e
    src_p = jnp.concatenate(
        [src.astype(jnp.int32), jnp.zeros((pad,), jnp.int32)]).reshape(NW * CPW, CH)
    dst_p = jnp.concatenate(
        [dst.astype(jnp.int32), jnp.full((pad,), N, jnp.int32)]).reshape(NW * CPW, CH)
    zeros = jnp.zeros((ZRPT, H), jnp.float32)

    h = _embed(x.astype(jnp.int32), node_emb)
    for i in range(NLAYER):
        m = _get_msg_kernel()(src_p, dst_p, zeros, h)
        m2 = m.reshape(NC, ACC_ROWS, H)[:, :N]
        h = _dense(h, m2[0], m2[1], conv_W1[i], conv_b1[i], conv_W2[i], conv_b2[i])
    return _pool_mlp(h, batch.astype(jnp.int32), mlp_W1, mlp_b1, mlp_W2, mlp_b2,
                     mlp_W3, mlp_b3)


# speedup vs baseline: 3.2815x; 1.1207x over previous
import functools

import jax
import jax.numpy as jnp
from jax import lax
from jax.experimental import pallas as pl
from jax.experimental.pallas import tpu as pltpu
from jax.experimental.pallas import tpu_sc as plsc

N = 10000
H = 128
NG = 64
NODE_VOCAB = 28
NLAYER = 3

NC = 2
NS = 16
NW = NC * NS
CH = 128
CPW = 80
EPW = CPW * CH
EPAD = NW * EPW
ACC_ROWS = 10112
ZRPT = ACC_ROWS // NS
GCH = 40

@functools.lru_cache(maxsize=None)
def _get_msg_kernel():
    mesh = plsc.VectorSubcoreMesh(core_axis_name="c", subcore_axis_name="s")

    @functools.partial(
        pl.kernel,
        mesh=mesh,
        out_type=jax.ShapeDtypeStruct((NC * ACC_ROWS, H), jnp.float32),
        scratch_types=[
            pltpu.VMEM((GCH, CH), jnp.int32),
            pltpu.VMEM((GCH, CH), jnp.int32),
            pltpu.VMEM((2, CH, H), jnp.float32),
            pltpu.VMEM_SHARED((ACC_ROWS, H), jnp.float32),
            pltpu.SemaphoreType.DMA,
            pltpu.SemaphoreType.DMA,
        ],
    )
    def _msg_kernel(src_hbm, dst_hbm, zero_hbm, h_hbm, out_hbm,
                    sidx, didx, rows, acc, sem0, sem1):
        c = lax.axis_index("c")
        s = lax.axis_index("s")
        pltpu.sync_copy(zero_hbm, acc.at[pl.ds(s * ZRPT, ZRPT)])
        plsc.subcore_barrier()

        wid = s * NC + c
        sems = (sem0, sem1)
        for gb in range(0, CPW, GCH):
            pltpu.sync_copy(src_hbm.at[pl.ds(wid * CPW + gb, GCH)], sidx)
            pltpu.sync_copy(dst_hbm.at[pl.ds(wid * CPW + gb, GCH)], didx)
            pltpu.async_copy(h_hbm.at[sidx.at[0]], rows.at[0], sem0)
            pltpu.async_copy(h_hbm.at[sidx.at[1]], rows.at[1], sem1)

            @pl.loop(0, GCH, step=2)
            def _(g):
                for b in range(2):
                    j = g + b
                    rb = rows.at[b]
                    pltpu.make_async_copy(h_hbm.at[sidx.at[j]], rb, sems[b]).wait()
                    pltpu.sync_copy(rb, acc.at[didx.at[j]], add=True)

                    @pl.when(j + 2 < GCH)
                    def _():
                        pltpu.async_copy(h_hbm.at[sidx.at[j + 2]], rb, sems[b])
        plsc.subcore_barrier()
        base = c * ACC_ROWS + s * ZRPT
        pltpu.sync_copy(acc.at[pl.ds(s * ZRPT, ZRPT)],
                        out_hbm.at[pl.ds(base, ZRPT)])

    return _msg_kernel


def _embed_body(x_ref, emb_ref, o_ref):
    xv = x_ref[...]
    vid = lax.broadcasted_iota(jnp.int32, (NODE_VOCAB, N), 0)
    onehot = (vid == xv).astype(jnp.float32)
    o_ref[...] = lax.dot_general(
        onehot, emb_ref[...], (((0,), (0,)), ((), ())),
        precision=lax.Precision.HIGHEST,
        preferred_element_type=jnp.float32)


def _embed(x, node_emb):
    return pl.pallas_call(
        _embed_body,
        out_shape=jax.ShapeDtypeStruct((N, H), jnp.float32),
    )(x.reshape(1, N), node_emb)


def _dense_body(h_ref, m0_ref, m1_ref, w1_ref, b1_ref, w2_ref, b2_ref, o_ref):
    xv = h_ref[...] + m0_ref[...] + m1_ref[...]
    z = jnp.dot(xv, w1_ref[...], preferred_element_type=jnp.float32) + b1_ref[...]
    z = jnp.maximum(z, 0.0)
    o_ref[...] = jnp.dot(z, w2_ref[...], preferred_element_type=jnp.float32) + b2_ref[...]


def _dense(h, m0, m1, w1, b1, w2, b2):
    blk = 2000
    grid = N // blk
    row = lambda i: (i, 0)
    full = lambda i: (0, 0)
    return pl.pallas_call(
        _dense_body,
        grid=(grid,),
        in_specs=[
            pl.BlockSpec((blk, H), row),
            pl.BlockSpec((blk, H), row),
            pl.BlockSpec((blk, H), row),
            pl.BlockSpec((H, H), full),
            pl.BlockSpec((1, H), full),
            pl.BlockSpec((H, H), full),
            pl.BlockSpec((1, H), full),
        ],
        out_specs=pl.BlockSpec((blk, H), row),
        out_shape=jax.ShapeDtypeStruct((N, H), jnp.float32),
    )(h, m0, m1, w1, b1.reshape(1, H), w2, b2.reshape(1, H))


def _pool_body(h_ref, bat_ref, w1_ref, b1_ref, w2_ref, b2_ref, w3_ref, b3_ref,
               o_ref):
    bat = bat_ref[...]
    gid = lax.broadcasted_iota(jnp.int32, (NG, N), 0)
    onehot = (gid == bat).astype(jnp.float32)
    pooled = jnp.dot(onehot, h_ref[...], precision=lax.Precision.HIGHEST,
                     preferred_element_type=jnp.float32)
    a = jnp.maximum(jnp.dot(pooled, w1_ref[...], preferred_element_type=jnp.float32)
                    + b1_ref[...], 0.0)
    a = jnp.maximum(jnp.dot(a, w2_ref[...], preferred_element_type=jnp.float32)
                    + b2_ref[...], 0.0)
    o_ref[...] = jnp.dot(a, w3_ref[...], preferred_element_type=jnp.float32) + b3_ref[...]


def _pool_mlp(h, batch, w1, b1, w2, b2, w3, b3):
    return pl.pallas_call(
        _pool_body,
        out_shape=jax.ShapeDtypeStruct((NG, 1), jnp.float32),
    )(h, batch.reshape(1, N), w1, b1.reshape(1, H // 2), w2,
      b2.reshape(1, H // 4), w3, b3.reshape(1, 1))


def kernel(x, edge_index, edge_attr, batch, node_emb, edge_emb,
           conv_W1, conv_b1, conv_W2, conv_b2,
           mlp_W1, mlp_b1, mlp_W2, mlp_b2, mlp_W3, mlp_b3):
    src = edge_index[0]
    dst = edge_index[1]
    e = src.shape[0]
    pad = EPAD - e
    src_p = jnp.concatenate(
        [src.astype(jnp.int32), jnp.zeros((pad,), jnp.int32)]).reshape(NW * CPW, CH)
    dst_p = jnp.concatenate(
        [dst.astype(jnp.int32), jnp.full((pad,), N, jnp.int32)]).reshape(NW * CPW, CH)
    zeros = jnp.zeros((ZRPT, H), jnp.float32)

    h = _embed(x.astype(jnp.int32), node_emb)
    for i in range(NLAYER):
        m = _get_msg_kernel()(src_p, dst_p, zeros, h)
        m2 = m.reshape(NC, ACC_ROWS, H)[:, :N]
        h = _dense(h, m2[0], m2[1], conv_W1[i], conv_b1[i], conv_W2[i], conv_b2[i])
    return _pool_mlp(h, batch.astype(jnp.int32), mlp_W1, mlp_b1, mlp_W2, mlp_b2,
                     mlp_W3, mlp_b3)


# speedup vs baseline: 4.3710x; 1.3320x over previous
import functools

import jax
import jax.numpy as jnp
from jax import lax
from jax.experimental import pallas as pl
from jax.experimental.pallas import tpu as pltpu
from jax.experimental.pallas import tpu_sc as plsc

N = 10000
H = 128
NG = 64
NODE_VOCAB = 28
NLAYER = 3

NC = 2
NS = 16
NW = NC * NS
CH = 128
CPW = 80
EPW = CPW * CH
EPAD = NW * EPW
ACC_ROWS = 10112
ZRPT = ACC_ROWS // NS
GCH = 40

@functools.lru_cache(maxsize=None)
def _get_msg_kernel():
    mesh = plsc.VectorSubcoreMesh(core_axis_name="c", subcore_axis_name="s")

    @functools.partial(
        pl.kernel,
        mesh=mesh,
        out_type=jax.ShapeDtypeStruct((NC * ACC_ROWS, H), jnp.float32),
        scratch_types=[
            pltpu.VMEM((GCH, CH), jnp.int32),
            pltpu.VMEM((GCH, CH), jnp.int32),
            pltpu.VMEM((2, CH, H), jnp.float32),
            pltpu.VMEM_SHARED((ACC_ROWS, H), jnp.float32),
            pltpu.SemaphoreType.DMA,
            pltpu.SemaphoreType.DMA,
        ],
    )
    def _msg_kernel(src_hbm, dst_hbm, zero_hbm, h_hbm, out_hbm,
                    sidx, didx, rows, acc, sem0, sem1):
        c = lax.axis_index("c")
        s = lax.axis_index("s")
        pltpu.sync_copy(zero_hbm, acc.at[pl.ds(s * ZRPT, ZRPT)])
        plsc.subcore_barrier()

        wid = s * NC + c
        sems = (sem0, sem1)
        for gb in range(0, CPW, GCH):
            pltpu.sync_copy(src_hbm.at[pl.ds(wid * CPW + gb, GCH)], sidx)
            pltpu.sync_copy(dst_hbm.at[pl.ds(wid * CPW + gb, GCH)], didx)
            pltpu.async_copy(h_hbm.at[sidx.at[0]], rows.at[0], sem0)
            pltpu.async_copy(h_hbm.at[sidx.at[1]], rows.at[1], sem1)

            @pl.loop(0, GCH, step=2)
            def _(g):
                for b in range(2):
                    j = g + b
                    rb = rows.at[b]
                    pltpu.make_async_copy(h_hbm.at[sidx.at[j]], rb, sems[b]).wait()
                    pltpu.sync_copy(rb, acc.at[didx.at[j]], add=True)

                    @pl.when(j + 2 < GCH)
                    def _():
                        pltpu.async_copy(h_hbm.at[sidx.at[j + 2]], rb, sems[b])
        plsc.subcore_barrier()
        base = c * ACC_ROWS + s * ZRPT
        pltpu.sync_copy(acc.at[pl.ds(s * ZRPT, ZRPT)],
                        out_hbm.at[pl.ds(base, ZRPT)])

    return _msg_kernel


VPAD = 32


@functools.lru_cache(maxsize=None)
def _get_cnt_kernel():
    mesh = plsc.VectorSubcoreMesh(core_axis_name="c", subcore_axis_name="s")

    ZV = ZRPT * VPAD

    @functools.partial(
        pl.kernel,
        mesh=mesh,
        out_type=jax.ShapeDtypeStruct((NC * ACC_ROWS * VPAD,), jnp.float32),
        scratch_types=[
            pltpu.VMEM((GCH, CH), jnp.int32),
            pltpu.VMEM((GCH, CH), jnp.int32),
            pltpu.VMEM((2, CH), jnp.int32),
            pltpu.VMEM((CH,), jnp.int32),
            pltpu.VMEM((CH,), jnp.float32),
            pltpu.VMEM_SHARED((ACC_ROWS * VPAD,), jnp.float32),
            pltpu.SemaphoreType.DMA,
            pltpu.SemaphoreType.DMA,
        ],
    )
    def _cnt_kernel(src_hbm, dst_hbm, zero_hbm, x_hbm, out_hbm,
                    sidx, didx, xs, idxb, ones, acc, xg0, xg1):
        c = lax.axis_index("c")
        s = lax.axis_index("s")
        for k in range(CH // 16):
            ones[pl.ds(k * 16, 16)] = jnp.full((16,), 1.0, jnp.float32)
        pltpu.sync_copy(zero_hbm, acc.at[pl.ds(s * ZV, ZV)])
        plsc.subcore_barrier()
        wid = s * NC + c
        xgs = (xg0, xg1)
        for gb in range(0, CPW, GCH):
            pltpu.sync_copy(src_hbm.at[pl.ds(wid * CPW + gb, GCH)], sidx)
            pltpu.sync_copy(dst_hbm.at[pl.ds(wid * CPW + gb, GCH)], didx)
            pltpu.async_copy(x_hbm.at[sidx.at[0]], xs.at[0], xg0)
            pltpu.async_copy(x_hbm.at[sidx.at[1]], xs.at[1], xg1)

            @pl.loop(0, GCH, step=2)
            def _(g):
                for b in range(2):
                    j = g + b
                    pltpu.make_async_copy(
                        x_hbm.at[sidx.at[j]], xs.at[b], xgs[b]).wait()
                    for k in range(CH // 16):
                        sl = pl.ds(k * 16, 16)
                        idxb[sl] = didx[j, sl] * VPAD + xs[b, sl]

                    @pl.when(j + 2 < GCH)
                    def _():
                        pltpu.async_copy(x_hbm.at[sidx.at[j + 2]], xs.at[b],
                                         xgs[b])
                    pltpu.sync_copy(ones, acc.at[idxb], add=True)

        plsc.subcore_barrier()
        base = c * ACC_ROWS * VPAD + s * ZV
        pltpu.sync_copy(acc.at[pl.ds(s * ZV, ZV)],
                        out_hbm.at[pl.ds(base, ZV)])

    return _cnt_kernel


def _embed_body(x_ref, emb_ref, o_ref):
    xv = x_ref[...]
    vid = lax.broadcasted_iota(jnp.int32, (NODE_VOCAB, N), 0)
    onehot = (vid == xv).astype(jnp.float32)
    o_ref[...] = lax.dot_general(
        onehot, emb_ref[...], (((0,), (0,)), ((), ())),
        precision=lax.Precision.HIGHEST,
        preferred_element_type=jnp.float32)


def _embed(x, node_emb):
    return pl.pallas_call(
        _embed_body,
        out_shape=jax.ShapeDtypeStruct((N, H), jnp.float32),
    )(x.reshape(1, N), node_emb)


def _dense_body(h_ref, m0_ref, m1_ref, w1_ref, b1_ref, w2_ref, b2_ref, o_ref):
    xv = h_ref[...] + m0_ref[...] + m1_ref[...]
    z = jnp.dot(xv, w1_ref[...], preferred_element_type=jnp.float32) + b1_ref[...]
    z = jnp.maximum(z, 0.0)
    o_ref[...] = jnp.dot(z, w2_ref[...], preferred_element_type=jnp.float32) + b2_ref[...]


def _dense(h, m0, m1, w1, b1, w2, b2):
    blk = 2000
    grid = N // blk
    row = lambda i: (i, 0)
    full = lambda i: (0, 0)
    return pl.pallas_call(
        _dense_body,
        grid=(grid,),
        in_specs=[
            pl.BlockSpec((blk, H), row),
            pl.BlockSpec((blk, H), row),
            pl.BlockSpec((blk, H), row),
            pl.BlockSpec((H, H), full),
            pl.BlockSpec((1, H), full),
            pl.BlockSpec((H, H), full),
            pl.BlockSpec((1, H), full),
        ],
        out_specs=pl.BlockSpec((blk, H), row),
        out_shape=jax.ShapeDtypeStruct((N, H), jnp.float32),
    )(h, m0, m1, w1, b1.reshape(1, H), w2, b2.reshape(1, H))


def _dense1_body(h_ref, c0_ref, c1_ref, emb_ref, w1_ref, b1_ref, w2_ref,
                 b2_ref, o_ref):
    cnt = c0_ref[...] + c1_ref[...]
    m = jnp.dot(cnt, emb_ref[...], precision=lax.Precision.HIGHEST,
                preferred_element_type=jnp.float32)
    xv = h_ref[...] + m
    z = jnp.dot(xv, w1_ref[...], preferred_element_type=jnp.float32) + b1_ref[...]
    z = jnp.maximum(z, 0.0)
    o_ref[...] = jnp.dot(z, w2_ref[...], preferred_element_type=jnp.float32) + b2_ref[...]


def _dense1(h, c0, c1, emb32, w1, b1, w2, b2):
    blk = 2000
    grid = N // blk
    row = lambda i: (i, 0)
    full = lambda i: (0, 0)
    return pl.pallas_call(
        _dense1_body,
        grid=(grid,),
        in_specs=[
            pl.BlockSpec((blk, H), row),
            pl.BlockSpec((blk, VPAD), row),
            pl.BlockSpec((blk, VPAD), row),
            pl.BlockSpec((VPAD, H), full),
            pl.BlockSpec((H, H), full),
            pl.BlockSpec((1, H), full),
            pl.BlockSpec((H, H), full),
            pl.BlockSpec((1, H), full),
        ],
        out_specs=pl.BlockSpec((blk, H), row),
        out_shape=jax.ShapeDtypeStruct((N, H), jnp.float32),
    )(h, c0, c1, emb32, w1, b1.reshape(1, H), w2, b2.reshape(1, H))


def _pool_body(h_ref, bat_ref, w1_ref, b1_ref, w2_ref, b2_ref, w3_ref, b3_ref,
               o_ref):
    bat = bat_ref[...]
    gid = lax.broadcasted_iota(jnp.int32, (NG, N), 0)
    onehot = (gid == bat).astype(jnp.float32)
    pooled = jnp.dot(onehot, h_ref[...], precision=lax.Precision.HIGHEST,
                     preferred_element_type=jnp.float32)
    a = jnp.maximum(jnp.dot(pooled, w1_ref[...], preferred_element_type=jnp.float32)
                    + b1_ref[...], 0.0)
    a = jnp.maximum(jnp.dot(a, w2_ref[...], preferred_element_type=jnp.float32)
                    + b2_ref[...], 0.0)
    o_ref[...] = jnp.dot(a, w3_ref[...], preferred_element_type=jnp.float32) + b3_ref[...]


def _pool_mlp(h, batch, w1, b1, w2, b2, w3, b3):
    return pl.pallas_call(
        _pool_body,
        out_shape=jax.ShapeDtypeStruct((NG, 1), jnp.float32),
    )(h, batch.reshape(1, N), w1, b1.reshape(1, H // 2), w2,
      b2.reshape(1, H // 4), w3, b3.reshape(1, 1))


def kernel(x, edge_index, edge_attr, batch, node_emb, edge_emb,
           conv_W1, conv_b1, conv_W2, conv_b2,
           mlp_W1, mlp_b1, mlp_W2, mlp_b2, mlp_W3, mlp_b3):
    src = edge_index[0]
    dst = edge_index[1]
    e = src.shape[0]
    pad = EPAD - e
    src_p = jnp.concatenate(
        [src.astype(jnp.int32), jnp.zeros((pad,), jnp.int32)]).reshape(NW * CPW, CH)
    dst_pad = N + jnp.arange(pad, dtype=jnp.int32) % (ACC_ROWS - N)
    dst_p = jnp.concatenate(
        [dst.astype(jnp.int32), dst_pad]).reshape(NW * CPW, CH)
    zeros = jnp.zeros((ZRPT, H), jnp.float32)
    zeros_v = jnp.zeros((ZRPT * VPAD,), jnp.float32)
    emb32 = jnp.pad(node_emb, ((0, VPAD - NODE_VOCAB), (0, 0)))
    x1 = x.reshape(N).astype(jnp.int32)

    h = _embed(x.astype(jnp.int32), node_emb)
    cm = _get_cnt_kernel()(src_p, dst_p, zeros_v, x1)
    cm2 = cm.reshape(NC, ACC_ROWS, VPAD)[:, :N]
    h = _dense1(h, cm2[0], cm2[1], emb32,
                conv_W1[0], conv_b1[0], conv_W2[0], conv_b2[0])
    for i in range(1, NLAYER):
        m = _get_msg_kernel()(src_p, dst_p, zeros, h)
        m2 = m.reshape(NC, ACC_ROWS, H)[:, :N]
        h = _dense(h, m2[0], m2[1], conv_W1[i], conv_b1[i], conv_W2[i], conv_b2[i])
    return _pool_mlp(h, batch.astype(jnp.int32), mlp_W1, mlp_b1, mlp_W2, mlp_b2,
                     mlp_W3, mlp_b3)
